# baseline (speedup 1.0000x reference)
import functools
import jax
import jax.numpy as jnp
from jax.experimental import pallas as pl
from jax.experimental.pallas import tpu as pltpu

_JC = 8


def _body(emb_ref, att_ref, w_ref, out_ref, a_ref, b_ref):
    L, D = emb_ref.shape[1], emb_ref.shape[2]
    NL = w_ref.shape[0]
    jc = pl.program_id(1)
    neg_inf = jnp.float32(-jnp.inf)

    @pl.when(jc == 0)
    def _():
        e = emb_ref[0]
        a = jax.lax.dot_general(e, w_ref[:, :D], (((1,), (1,)), ((), ())),
                                preferred_element_type=jnp.float32)
        bv = jax.lax.dot_general(e, w_ref[:, D:], (((1,), (1,)), ((), ())),
                                 preferred_element_type=jnp.float32)
        attc = att_ref[0]
        a_ref[...] = jnp.where(attc > 0, a, neg_inf)
        b_ref[...] = jnp.where(attc > 0, bv, neg_inf)

    a = a_ref[...]
    bchunk = b_ref[pl.ds(jc * _JC, _JC), :]
    blk = a[None, :, :] + bchunk[:, None, :]
    jg = jc * _JC + jax.lax.broadcasted_iota(jnp.int32, (_JC, L, 1), 0)
    kg = jax.lax.broadcasted_iota(jnp.int32, (_JC, L, 1), 1)
    blk = jnp.where(jg == kg, neg_inf, blk)
    out_ref[0] = blk.reshape(_JC * L, NL)


def kernel(emb_sentences, att_sentences, W):
    B, L, D = emb_sentences.shape
    NL = W.shape[0]
    att_col = att_sentences.astype(jnp.float32).reshape(B, L, 1)
    return pl.pallas_call(
        functools.partial(_body),
        grid=(B, L // _JC),
        in_specs=[
            pl.BlockSpec((1, L, D), lambda b, jc: (b, 0, 0)),
            pl.BlockSpec((1, L, 1), lambda b, jc: (b, 0, 0)),
            pl.BlockSpec((NL, 2 * D), lambda b, jc: (0, 0)),
        ],
        out_specs=pl.BlockSpec((1, _JC * L, NL), lambda b, jc: (b, jc, 0)),
        out_shape=jax.ShapeDtypeStruct((B, L * L, NL), jnp.float32),
        scratch_shapes=[
            pltpu.VMEM((L, NL), jnp.float32),
            pltpu.VMEM((L, NL), jnp.float32),
        ],
    )(emb_sentences, att_col, W)


# speedup vs baseline: 1.8613x; 1.8613x over previous
import jax
import jax.numpy as jnp
from jax.experimental import pallas as pl
from jax.experimental.pallas import tpu as pltpu

_NCH = 4
_JC = 8


def _body(emb_ref, att_ref, w_ref, out_ref, buf_ref, sem):
    B = out_ref.shape[0]
    L, D = emb_ref.shape[1], emb_ref.shape[2]
    NL = w_ref.shape[0]
    LL = L * L
    CH = LL // _NCH
    b = pl.program_id(0)
    par = jax.lax.rem(b, 2)
    neg_inf = jnp.float32(-jnp.inf)

    @pl.when(b >= 2)
    def _():
        for c in range(_NCH):
            pltpu.make_async_copy(
                buf_ref.at[par, pl.ds(c * CH, CH), :],
                out_ref.at[b - 2, pl.ds(c * CH, CH), :],
                sem.at[par, c]).wait()

    e = emb_ref[0]
    a = jax.lax.dot_general(e, w_ref[:, :D], (((1,), (1,)), ((), ())),
                            preferred_element_type=jnp.float32)
    bv = jax.lax.dot_general(e, w_ref[:, D:], (((1,), (1,)), ((), ())),
                             preferred_element_type=jnp.float32)
    attc = att_ref[0]
    a = jnp.where(attc > 0, a, neg_inf)
    bv = jnp.where(attc > 0, bv, neg_inf)
    for jc in range(L // _JC):
        bchunk = bv[jc * _JC:(jc + 1) * _JC]
        blk = a[None, :, :] + bchunk[:, None, :]
        jg = jc * _JC + jax.lax.broadcasted_iota(jnp.int32, (_JC, L, 1), 0)
        kg = jax.lax.broadcasted_iota(jnp.int32, (_JC, L, 1), 1)
        blk = jnp.where(jg == kg, neg_inf, blk)
        buf_ref[par, pl.ds(jc * _JC * L, _JC * L), :] = blk.reshape(_JC * L, NL)

    for c in range(_NCH):
        pltpu.make_async_copy(
            buf_ref.at[par, pl.ds(c * CH, CH), :],
            out_ref.at[b, pl.ds(c * CH, CH), :],
            sem.at[par, c]).start()

    @pl.when(b == B - 1)
    def _():
        for c in range(_NCH):
            pltpu.make_async_copy(
                buf_ref.at[1 - par, pl.ds(c * CH, CH), :],
                out_ref.at[b - 1, pl.ds(c * CH, CH), :],
                sem.at[1 - par, c]).wait()
            pltpu.make_async_copy(
                buf_ref.at[par, pl.ds(c * CH, CH), :],
                out_ref.at[b, pl.ds(c * CH, CH), :],
                sem.at[par, c]).wait()


def kernel(emb_sentences, att_sentences, W):
    B, L, D = emb_sentences.shape
    NL = W.shape[0]
    att_col = att_sentences.astype(jnp.float32).reshape(B, L, 1)
    return pl.pallas_call(
        _body,
        grid=(B,),
        in_specs=[
            pl.BlockSpec((1, L, D), lambda b: (b, 0, 0)),
            pl.BlockSpec((1, L, 1), lambda b: (b, 0, 0)),
            pl.BlockSpec((NL, 2 * D), lambda b: (0, 0)),
        ],
        out_specs=pl.BlockSpec(memory_space=pl.ANY),
        out_shape=jax.ShapeDtypeStruct((B, L * L, NL), jnp.float32),
        scratch_shapes=[
            pltpu.VMEM((2, L * L, NL), jnp.float32),
            pltpu.SemaphoreType.DMA((2, _NCH)),
        ],
    )(emb_sentences, att_col, W)
